# baseline (speedup 1.0000x reference)
import functools

import jax
import jax.numpy as jnp
from jax import lax
from jax.experimental import pallas as pl
from jax.experimental.pallas import tpu as pltpu
from jax.experimental.pallas import tpu_sc as plsc

N = 10000
E = 320000
NP = 10240
EP = 327680
NC = 2
NS = 16
NW = NC * NS
CH = 128
CPW = EP // NW // CH
EPW = EP // NW
NPW = NP // NW
NPT = NP // NS

_MESH = plsc.VectorSubcoreMesh(core_axis_name="c", subcore_axis_name="s",
                               num_cores=NC, num_subcores=NS)
_SC_PARAMS = pltpu.CompilerParams(needs_layout_passes=False)
F32 = jnp.float32
I32 = jnp.int32


def _rsqrt(x):
    y = 1.0 / x
    for _ in range(12):
        y = y * (1.5 - 0.5 * x * y * y)
    return y


def _mm_body(x_ref, w_ref, o_ref):
    o_ref[...] = jnp.dot(x_ref[...], w_ref[...], preferred_element_type=F32)


def _matmul(x, w):
    return pl.pallas_call(
        _mm_body,
        out_shape=jax.ShapeDtypeStruct((N, w.shape[1]), F32),
    )(x, w)


@functools.partial(
    pl.kernel,
    out_type=jax.ShapeDtypeStruct((NC * NP,), F32),
    mesh=_MESH,
    compiler_params=_SC_PARAMS,
    scratch_types=[
        pltpu.VMEM((CPW, CH), I32),
        pltpu.VMEM((CPW, CH), F32),
        pltpu.VMEM_SHARED((NP,), F32),
        pltpu.SemaphoreType.DMA,
    ],
)
def _sc_deg(dst2d, ew2d, zeros1, out, dst_v, ew_v, deg_sh, sem):
    c = lax.axis_index("c")
    s = lax.axis_index("s")
    w = s * NC + c
    pltpu.sync_copy(dst2d.at[pl.ds(w * CPW, CPW)], dst_v)
    pltpu.sync_copy(ew2d.at[pl.ds(w * CPW, CPW)], ew_v)
    pltpu.sync_copy(zeros1.at[pl.ds(s * NPT, NPT)],
                    deg_sh.at[pl.ds(s * NPT, NPT)])
    plsc.subcore_barrier()

    RING = 8

    def fire(j, carry):
        @pl.when(j >= RING)
        def _():
            pltpu.make_async_copy(ew_v.at[j - RING],
                                  deg_sh.at[dst_v.at[j - RING]], sem).wait()
        pltpu.async_copy(ew_v.at[j], deg_sh.at[dst_v.at[j]], sem, add=True)
        return carry

    lax.fori_loop(0, CPW, fire, 0)

    def drain(j, carry):
        pltpu.make_async_copy(ew_v.at[j], deg_sh.at[dst_v.at[j]], sem).wait()
        return carry

    lax.fori_loop(CPW - RING, CPW, drain, 0)
    plsc.subcore_barrier()
    pltpu.sync_copy(deg_sh.at[pl.ds(s * NPT, NPT)],
                    out.at[pl.ds(c * NP + s * NPT, NPT)])


@functools.partial(
    pl.kernel,
    out_type=(jax.ShapeDtypeStruct((NP,), F32),
              jax.ShapeDtypeStruct((NP * 4,), F32)),
    mesh=_MESH,
    compiler_params=_SC_PARAMS,
    scratch_types=[
        pltpu.VMEM((NPW,), F32),
        pltpu.VMEM((NPW,), F32),
        pltpu.VMEM((NPW * 4,), F32),
        pltpu.VMEM((NPW,), F32),
        pltpu.VMEM((NPW * 4,), F32),
    ],
)
def _sc_dis_y(degp, xwf, dis_o, y_o, d0, d1, xv, dv, yv):
    c = lax.axis_index("c")
    s = lax.axis_index("s")
    w = s * NC + c
    nb = w * NPW
    pltpu.sync_copy(degp.at[pl.ds(nb, NPW)], d0)
    pltpu.sync_copy(degp.at[pl.ds(NP + nb, NPW)], d1)
    pltpu.sync_copy(xwf.at[pl.ds(nb * 4, NPW * 4)], xv)
    iota = lax.iota(I32, 16)
    for i in range(NPW // 16):
        f = pl.ds(i * 16, 16)
        deg = d0[f] + d1[f] + 1.0
        dv[f] = _rsqrt(deg)
    for i in range(NPW * 4 // 16):
        rep = lax.shift_right_logical(i * 16 + iota, 2)
        dis_rep = plsc.load_gather(dv, [rep])
        f = pl.ds(i * 16, 16)
        yv[f] = dis_rep * xv[f]
    pltpu.sync_copy(dv, dis_o.at[pl.ds(nb, NPW)])
    pltpu.sync_copy(yv, y_o.at[pl.ds(nb * 4, NPW * 4)])


_NB1 = 4


def _edge1_scratch():
    t = [pltpu.VMEM((EPW,), I32),
         pltpu.VMEM((EPW,), I32),
         pltpu.VMEM((EPW,), F32),
         pltpu.VMEM((NP * 4,), F32)]
    t += [pltpu.VMEM((CH,), F32) for _ in range(4 * _NB1)]
    t += [pltpu.VMEM((CH,), I32) for _ in range(4 * _NB1)]
    t += [pltpu.VMEM_SHARED((NP * 4,), F32)]
    t += [pltpu.SemaphoreType.DMA for _ in range(_NB1)]
    return t


@functools.partial(
    pl.kernel,
    out_type=jax.ShapeDtypeStruct((NC * NP * 4,), F32),
    mesh=_MESH,
    compiler_params=_SC_PARAMS,
    scratch_types=_edge1_scratch(),
)
def _sc_edge1(srcf, ewf, dstf, yf, zeros4, out, *scr):
    src_v, dst_v, ew_v, y_v = scr[:4]
    mflat = scr[4:4 + 4 * _NB1]
    iflat = scr[4 + 4 * _NB1:4 + 8 * _NB1]
    msgs = tuple(mflat[4 * b:4 * b + 4] for b in range(_NB1))
    idxs = tuple(iflat[4 * b:4 * b + 4] for b in range(_NB1))
    agg_sh = scr[4 + 8 * _NB1]
    sems = scr[5 + 8 * _NB1:]
    c = lax.axis_index("c")
    s = lax.axis_index("s")
    w = s * NC + c
    pltpu.sync_copy(srcf.at[pl.ds(w * EPW, EPW)], src_v)
    pltpu.sync_copy(dstf.at[pl.ds(w * EPW, EPW)], dst_v)
    pltpu.sync_copy(ewf.at[pl.ds(w * EPW, EPW)], ew_v)
    pltpu.sync_copy(yf, y_v)
    pltpu.sync_copy(zeros4.at[pl.ds(s * NPT * 4, NPT * 4)],
                    agg_sh.at[pl.ds(s * NPT * 4, NPT * 4)])
    plsc.subcore_barrier()

    def body(jj, carry):
        for b in range(_NB1):
            j = jj * _NB1 + b

            @pl.when(jj > 0)
            def _():
                for col in range(4):
                    pltpu.make_async_copy(
                        msgs[b][col], agg_sh.at[idxs[b][col]], sems[b]).wait()

            eb = j * CH
            for g in range(CH // 16):
                f = pl.ds(eb + g * 16, 16)
                o = pl.ds(g * 16, 16)
                src16 = src_v[f]
                dst16 = dst_v[f]
                ew16 = ew_v[f]
                s4 = src16 * 4
                d4 = dst16 * 4
                for col in range(4):
                    yg = plsc.load_gather(y_v, [s4 + col])
                    msgs[b][col][o] = yg * ew16
                    idxs[b][col][o] = d4 + col
            for col in range(4):
                pltpu.async_copy(msgs[b][col], agg_sh.at[idxs[b][col]],
                                 sems[b], add=True)
        return carry

    lax.fori_loop(0, CPW // _NB1, body, 0)
    for b in range(_NB1):
        for col in range(4):
            pltpu.make_async_copy(
                msgs[b][col], agg_sh.at[idxs[b][col]], sems[b]).wait()
    plsc.subcore_barrier()
    pltpu.sync_copy(agg_sh.at[pl.ds(s * NPT * 4, NPT * 4)],
                    out.at[pl.ds(c * NP * 4 + s * NPT * 4, NPT * 4)])


@functools.partial(
    pl.kernel,
    out_type=jax.ShapeDtypeStruct((NP,), F32),
    mesh=_MESH,
    compiler_params=_SC_PARAMS,
    scratch_types=[
        pltpu.VMEM((NPW * 4,), F32),
        pltpu.VMEM((NPW * 4,), F32),
        pltpu.VMEM((NPW * 4,), F32),
        pltpu.VMEM((NPW,), F32),
        pltpu.VMEM((NPW * 4,), F32),
        pltpu.VMEM((NPW,), F32),
        pltpu.VMEM((16,), F32),
        pltpu.VMEM((16,), F32),
    ],
)
def _sc_hz(p1, yf, disf, b1rep, w2rep, z_o,
           a0, a1, yv, dv, hw, zv, b1v, w2v):
    c = lax.axis_index("c")
    s = lax.axis_index("s")
    w = s * NC + c
    nb = w * NPW
    pltpu.sync_copy(p1.at[pl.ds(nb * 4, NPW * 4)], a0)
    pltpu.sync_copy(p1.at[pl.ds(NP * 4 + nb * 4, NPW * 4)], a1)
    pltpu.sync_copy(yf.at[pl.ds(nb * 4, NPW * 4)], yv)
    pltpu.sync_copy(disf.at[pl.ds(nb, NPW)], dv)
    pltpu.sync_copy(b1rep, b1v)
    pltpu.sync_copy(w2rep, w2v)
    iota = lax.iota(I32, 16)
    b1x = b1v[pl.ds(0, 16)]
    w2x = w2v[pl.ds(0, 16)]
    for i in range(NPW * 4 // 16):
        f = pl.ds(i * 16, 16)
        rep = lax.shift_right_logical(i * 16 + iota, 2)
        dis_rep = plsc.load_gather(dv, [rep])
        h = dis_rep * (a0[f] + a1[f] + yv[f]) + b1x
        h = jnp.maximum(h, 0.0)
        hw[f] = h * w2x
    for i in range(NPW // 16):
        f = pl.ds(i * 16, 16)
        base = (i * 16 + iota) * 4
        t = plsc.load_gather(hw, [base])
        t = t + plsc.load_gather(hw, [base + 1])
        t = t + plsc.load_gather(hw, [base + 2])
        t = t + plsc.load_gather(hw, [base + 3])
        zv[f] = dv[f] * t
    pltpu.sync_copy(zv, z_o.at[pl.ds(nb, NPW)])


@functools.partial(
    pl.kernel,
    out_type=jax.ShapeDtypeStruct((NC * NP,), F32),
    mesh=_MESH,
    compiler_params=_SC_PARAMS,
    scratch_types=[
        pltpu.VMEM((EPW,), I32),
        pltpu.VMEM((CPW, CH), I32),
        pltpu.VMEM((EPW,), F32),
        pltpu.VMEM((NP,), F32),
        pltpu.VMEM((CH,), F32),
        pltpu.VMEM((CH,), F32),
        pltpu.VMEM_SHARED((NP,), F32),
        pltpu.SemaphoreType.DMA,
        pltpu.SemaphoreType.DMA,
    ],
)
def _sc_edge2(srcf, ewf, dst2d, zf, zeros1, out,
              src_v, dst_v, ew_v, z_v, mb0, mb1, agg_sh, sem0, sem1):
    c = lax.axis_index("c")
    s = lax.axis_index("s")
    w = s * NC + c
    pltpu.sync_copy(srcf.at[pl.ds(w * EPW, EPW)], src_v)
    pltpu.sync_copy(dst2d.at[pl.ds(w * CPW, CPW)], dst_v)
    pltpu.sync_copy(ewf.at[pl.ds(w * EPW, EPW)], ew_v)
    pltpu.sync_copy(zf, z_v)
    pltpu.sync_copy(zeros1.at[pl.ds(s * NPT, NPT)],
                    agg_sh.at[pl.ds(s * NPT, NPT)])
    plsc.subcore_barrier()
    msgs = (mb0, mb1)
    sems = (sem0, sem1)

    def body(jj, carry):
        for b in range(2):
            j = jj * 2 + b

            @pl.when(jj > 0)
            def _():
                pltpu.make_async_copy(msgs[b], agg_sh.at[dst_v.at[j]],
                                      sems[b]).wait()

            eb = j * CH
            for g in range(CH // 16):
                f = pl.ds(eb + g * 16, 16)
                zg = plsc.load_gather(z_v, [src_v[f]])
                msgs[b][pl.ds(g * 16, 16)] = zg * ew_v[f]
            pltpu.async_copy(msgs[b], agg_sh.at[dst_v.at[j]], sems[b],
                             add=True)
        return carry

    lax.fori_loop(0, CPW // 2, body, 0)
    for b in range(2):
        pltpu.make_async_copy(msgs[b], agg_sh.at[dst_v.at[0]], sems[b]).wait()
    plsc.subcore_barrier()
    pltpu.sync_copy(agg_sh.at[pl.ds(s * NPT, NPT)],
                    out.at[pl.ds(c * NP + s * NPT, NPT)])


@functools.partial(
    pl.kernel,
    out_type=jax.ShapeDtypeStruct((NP,), F32),
    mesh=_MESH,
    compiler_params=_SC_PARAMS,
    scratch_types=[
        pltpu.VMEM((NPW,), F32),
        pltpu.VMEM((NPW,), F32),
        pltpu.VMEM((NPW,), F32),
        pltpu.VMEM((NPW,), F32),
        pltpu.VMEM((NPW,), F32),
        pltpu.VMEM((16,), F32),
        pltpu.VMEM((16,), F32),
        pltpu.VMEM((16,), F32),
    ],
)
def _sc_fin(p2, zf, disf, b2s, wls, bls, res_o, a0, a1, zv, dv, rv,
            b2v, wlv, blv):
    c = lax.axis_index("c")
    s = lax.axis_index("s")
    w = s * NC + c
    nb = w * NPW
    pltpu.sync_copy(p2.at[pl.ds(nb, NPW)], a0)
    pltpu.sync_copy(p2.at[pl.ds(NP + nb, NPW)], a1)
    pltpu.sync_copy(zf.at[pl.ds(nb, NPW)], zv)
    pltpu.sync_copy(disf.at[pl.ds(nb, NPW)], dv)
    pltpu.sync_copy(b2s, b2v)
    pltpu.sync_copy(wls, wlv)
    pltpu.sync_copy(bls, blv)
    b2x = b2v[pl.ds(0, 16)]
    wlx = wlv[pl.ds(0, 16)]
    blx = blv[pl.ds(0, 16)]
    for i in range(NPW // 16):
        f = pl.ds(i * 16, 16)
        o2 = dv[f] * (a0[f] + a1[f] + zv[f]) + b2x
        t = o2 * wlx + blx
        rv[f] = 1.0 / (1.0 + jnp.exp(-t))
    pltpu.sync_copy(rv, res_o.at[pl.ds(nb, NPW)])


def kernel(x, edge_index, edge_attr, W1, b1, W2, b2, Wl, bl):
    src = edge_index[0]
    dst = edge_index[1]
    npad = EP - E
    pad_idx = (jnp.arange(npad, dtype=jnp.int32) * 37) % N
    srcp = jnp.concatenate([src, pad_idx])
    dstp = jnp.concatenate([dst, pad_idx])
    ewp = jnp.concatenate([edge_attr, jnp.zeros((npad,), F32)])
    dst2d = dstp.reshape(EP // CH, CH)
    ew2d = ewp.reshape(EP // CH, CH)

    xw = _matmul(x, W1)
    xwf = jnp.pad(xw, ((0, NP - N), (0, 0))).reshape(NP * 4)

    zeros1 = jnp.zeros((NP,), F32)
    zeros4 = jnp.zeros((NP * 4,), F32)
    b1rep = jnp.tile(b1, 4)
    w2rep = jnp.tile(W2[:, 0], 4)
    b2s = jnp.full((16,), b2[0], F32)
    wls = jnp.full((16,), Wl[0, 0], F32)
    bls = jnp.full((16,), bl[0], F32)

    degp = _sc_deg(dst2d, ew2d, zeros1)
    dis, yf = _sc_dis_y(degp, xwf)
    p1f = _sc_edge1(srcp, ewp, dstp, yf, zeros4)
    z = _sc_hz(p1f, yf, dis, b1rep, w2rep)
    p2 = _sc_edge2(srcp, ewp, dst2d, z, zeros1)
    res = _sc_fin(p2, z, dis, b2s, wls, bls)
    return res[:N, None]


# speedup vs baseline: 1.0172x; 1.0172x over previous
import functools

import jax
import jax.numpy as jnp
from jax import lax
from jax.experimental import pallas as pl
from jax.experimental.pallas import tpu as pltpu
from jax.experimental.pallas import tpu_sc as plsc

N = 10000
E = 320000
NP = 10240
EP = 327680
NC = 2
NS = 16
NW = NC * NS
CH = 128
CPW = EP // NW // CH
EPW = EP // NW
NPW = NP // NW
NPT = NP // NS

_MESH = plsc.VectorSubcoreMesh(core_axis_name="c", subcore_axis_name="s",
                               num_cores=NC, num_subcores=NS)
_SC_PARAMS = pltpu.CompilerParams(needs_layout_passes=False)
F32 = jnp.float32
I32 = jnp.int32


def _rsqrt(x):
    y = 1.0 / x
    for _ in range(12):
        y = y * (1.5 - 0.5 * x * y * y)
    return y


def _mm_body(x_ref, w_ref, o_ref):
    o_ref[...] = jnp.dot(x_ref[...], w_ref[...], preferred_element_type=F32)


def _matmul(x, w):
    return pl.pallas_call(
        _mm_body,
        out_shape=jax.ShapeDtypeStruct((N, w.shape[1]), F32),
    )(x, w)


@functools.partial(
    pl.kernel,
    out_type=jax.ShapeDtypeStruct((NC * NP,), F32),
    mesh=_MESH,
    compiler_params=_SC_PARAMS,
    scratch_types=[
        pltpu.VMEM((CPW, CH), I32),
        pltpu.VMEM((CPW, CH), F32),
        pltpu.VMEM_SHARED((NP,), F32),
        pltpu.SemaphoreType.DMA,
    ],
)
def _sc_deg(dst2d, ew2d, zeros1, out, dst_v, ew_v, deg_sh, sem):
    c = lax.axis_index("c")
    s = lax.axis_index("s")
    w = s * NC + c
    pltpu.sync_copy(dst2d.at[pl.ds(w * CPW, CPW)], dst_v)
    pltpu.sync_copy(ew2d.at[pl.ds(w * CPW, CPW)], ew_v)
    pltpu.sync_copy(zeros1.at[pl.ds(s * NPT, NPT)],
                    deg_sh.at[pl.ds(s * NPT, NPT)])
    plsc.subcore_barrier()

    def fire(j, carry):
        pltpu.async_copy(ew_v.at[j], deg_sh.at[dst_v.at[j]], sem, add=True)
        return carry

    lax.fori_loop(0, CPW, fire, 0)

    def drain(j, carry):
        pltpu.make_async_copy(ew_v.at[j], deg_sh.at[dst_v.at[j]], sem).wait()
        return carry

    lax.fori_loop(0, CPW, drain, 0)
    plsc.subcore_barrier()
    pltpu.sync_copy(deg_sh.at[pl.ds(s * NPT, NPT)],
                    out.at[pl.ds(c * NP + s * NPT, NPT)])


@functools.partial(
    pl.kernel,
    out_type=(jax.ShapeDtypeStruct((NP,), F32),
              jax.ShapeDtypeStruct((NP * 4,), F32)),
    mesh=_MESH,
    compiler_params=_SC_PARAMS,
    scratch_types=[
        pltpu.VMEM((NPW,), F32),
        pltpu.VMEM((NPW,), F32),
        pltpu.VMEM((NPW * 4,), F32),
        pltpu.VMEM((NPW,), F32),
        pltpu.VMEM((NPW * 4,), F32),
    ],
)
def _sc_dis_y(degp, xwf, dis_o, y_o, d0, d1, xv, dv, yv):
    c = lax.axis_index("c")
    s = lax.axis_index("s")
    w = s * NC + c
    nb = w * NPW
    pltpu.sync_copy(degp.at[pl.ds(nb, NPW)], d0)
    pltpu.sync_copy(degp.at[pl.ds(NP + nb, NPW)], d1)
    pltpu.sync_copy(xwf.at[pl.ds(nb * 4, NPW * 4)], xv)
    iota = lax.iota(I32, 16)
    for i in range(NPW // 16):
        f = pl.ds(i * 16, 16)
        deg = d0[f] + d1[f] + 1.0
        dv[f] = _rsqrt(deg)
    for i in range(NPW * 4 // 16):
        rep = lax.shift_right_logical(i * 16 + iota, 2)
        dis_rep = plsc.load_gather(dv, [rep])
        f = pl.ds(i * 16, 16)
        yv[f] = dis_rep * xv[f]
    pltpu.sync_copy(dv, dis_o.at[pl.ds(nb, NPW)])
    pltpu.sync_copy(yv, y_o.at[pl.ds(nb * 4, NPW * 4)])


@functools.partial(
    pl.kernel,
    out_type=jax.ShapeDtypeStruct((NC * NP * 4,), F32),
    mesh=_MESH,
    compiler_params=_SC_PARAMS,
    scratch_types=(
        [pltpu.VMEM((EPW,), I32),
         pltpu.VMEM((EPW,), I32),
         pltpu.VMEM((EPW,), F32),
         pltpu.VMEM((NP * 4,), F32)]
        + [pltpu.VMEM((CH,), F32)] * 8
        + [pltpu.VMEM((CH,), I32)] * 8
        + [pltpu.VMEM_SHARED((NP * 4,), F32),
           pltpu.SemaphoreType.DMA,
           pltpu.SemaphoreType.DMA]
    ),
)
def _sc_edge1(srcf, ewf, dstf, yf, zeros4, out,
              src_v, dst_v, ew_v, y_v,
              m00, m01, m02, m03, m10, m11, m12, m13,
              i00, i01, i02, i03, i10, i11, i12, i13,
              agg_sh, sem0, sem1):
    c = lax.axis_index("c")
    s = lax.axis_index("s")
    w = s * NC + c
    pltpu.sync_copy(srcf.at[pl.ds(w * EPW, EPW)], src_v)
    pltpu.sync_copy(dstf.at[pl.ds(w * EPW, EPW)], dst_v)
    pltpu.sync_copy(ewf.at[pl.ds(w * EPW, EPW)], ew_v)
    pltpu.sync_copy(yf, y_v)
    pltpu.sync_copy(zeros4.at[pl.ds(s * NPT * 4, NPT * 4)],
                    agg_sh.at[pl.ds(s * NPT * 4, NPT * 4)])
    plsc.subcore_barrier()
    msgs = ((m00, m01, m02, m03), (m10, m11, m12, m13))
    idxs = ((i00, i01, i02, i03), (i10, i11, i12, i13))
    sems = (sem0, sem1)

    def body(jj, carry):
        for b in range(2):
            j = jj * 2 + b

            @pl.when(jj > 0)
            def _():
                for col in range(4):
                    pltpu.make_async_copy(
                        msgs[b][col], agg_sh.at[idxs[b][col]], sems[b]).wait()

            eb = j * CH
            for g in range(CH // 16):
                f = pl.ds(eb + g * 16, 16)
                o = pl.ds(g * 16, 16)
                src16 = src_v[f]
                dst16 = dst_v[f]
                ew16 = ew_v[f]
                s4 = src16 * 4
                d4 = dst16 * 4
                for col in range(4):
                    yg = plsc.load_gather(y_v, [s4 + col])
                    msgs[b][col][o] = yg * ew16
                    idxs[b][col][o] = d4 + col
            for col in range(4):
                pltpu.async_copy(msgs[b][col], agg_sh.at[idxs[b][col]],
                                 sems[b], add=True)
        return carry

    lax.fori_loop(0, CPW // 2, body, 0)
    for b in range(2):
        for col in range(4):
            pltpu.make_async_copy(
                msgs[b][col], agg_sh.at[idxs[b][col]], sems[b]).wait()
    plsc.subcore_barrier()
    pltpu.sync_copy(agg_sh.at[pl.ds(s * NPT * 4, NPT * 4)],
                    out.at[pl.ds(c * NP * 4 + s * NPT * 4, NPT * 4)])


@functools.partial(
    pl.kernel,
    out_type=jax.ShapeDtypeStruct((NP,), F32),
    mesh=_MESH,
    compiler_params=_SC_PARAMS,
    scratch_types=[
        pltpu.VMEM((NPW * 4,), F32),
        pltpu.VMEM((NPW * 4,), F32),
        pltpu.VMEM((NPW * 4,), F32),
        pltpu.VMEM((NPW,), F32),
        pltpu.VMEM((NPW * 4,), F32),
        pltpu.VMEM((NPW,), F32),
        pltpu.VMEM((16,), F32),
        pltpu.VMEM((16,), F32),
    ],
)
def _sc_hz(p1, yf, disf, b1rep, w2rep, z_o,
           a0, a1, yv, dv, hw, zv, b1v, w2v):
    c = lax.axis_index("c")
    s = lax.axis_index("s")
    w = s * NC + c
    nb = w * NPW
    pltpu.sync_copy(p1.at[pl.ds(nb * 4, NPW * 4)], a0)
    pltpu.sync_copy(p1.at[pl.ds(NP * 4 + nb * 4, NPW * 4)], a1)
    pltpu.sync_copy(yf.at[pl.ds(nb * 4, NPW * 4)], yv)
    pltpu.sync_copy(disf.at[pl.ds(nb, NPW)], dv)
    pltpu.sync_copy(b1rep, b1v)
    pltpu.sync_copy(w2rep, w2v)
    iota = lax.iota(I32, 16)
    b1x = b1v[pl.ds(0, 16)]
    w2x = w2v[pl.ds(0, 16)]
    for i in range(NPW * 4 // 16):
        f = pl.ds(i * 16, 16)
        rep = lax.shift_right_logical(i * 16 + iota, 2)
        dis_rep = plsc.load_gather(dv, [rep])
        h = dis_rep * (a0[f] + a1[f] + yv[f]) + b1x
        h = jnp.maximum(h, 0.0)
        hw[f] = h * w2x
    for i in range(NPW // 16):
        f = pl.ds(i * 16, 16)
        base = (i * 16 + iota) * 4
        t = plsc.load_gather(hw, [base])
        t = t + plsc.load_gather(hw, [base + 1])
        t = t + plsc.load_gather(hw, [base + 2])
        t = t + plsc.load_gather(hw, [base + 3])
        zv[f] = dv[f] * t
    pltpu.sync_copy(zv, z_o.at[pl.ds(nb, NPW)])


@functools.partial(
    pl.kernel,
    out_type=jax.ShapeDtypeStruct((NC * NP,), F32),
    mesh=_MESH,
    compiler_params=_SC_PARAMS,
    scratch_types=[
        pltpu.VMEM((EPW,), I32),
        pltpu.VMEM((CPW, CH), I32),
        pltpu.VMEM((EPW,), F32),
        pltpu.VMEM((NP,), F32),
        pltpu.VMEM((CH,), F32),
        pltpu.VMEM((CH,), F32),
        pltpu.VMEM_SHARED((NP,), F32),
        pltpu.SemaphoreType.DMA,
        pltpu.SemaphoreType.DMA,
    ],
)
def _sc_edge2(srcf, ewf, dst2d, zf, zeros1, out,
              src_v, dst_v, ew_v, z_v, mb0, mb1, agg_sh, sem0, sem1):
    c = lax.axis_index("c")
    s = lax.axis_index("s")
    w = s * NC + c
    pltpu.sync_copy(srcf.at[pl.ds(w * EPW, EPW)], src_v)
    pltpu.sync_copy(dst2d.at[pl.ds(w * CPW, CPW)], dst_v)
    pltpu.sync_copy(ewf.at[pl.ds(w * EPW, EPW)], ew_v)
    pltpu.sync_copy(zf, z_v)
    pltpu.sync_copy(zeros1.at[pl.ds(s * NPT, NPT)],
                    agg_sh.at[pl.ds(s * NPT, NPT)])
    plsc.subcore_barrier()
    msgs = (mb0, mb1)
    sems = (sem0, sem1)

    def body(jj, carry):
        for b in range(2):
            j = jj * 2 + b

            @pl.when(jj > 0)
            def _():
                pltpu.make_async_copy(msgs[b], agg_sh.at[dst_v.at[j]],
                                      sems[b]).wait()

            eb = j * CH
            for g in range(CH // 16):
                f = pl.ds(eb + g * 16, 16)
                zg = plsc.load_gather(z_v, [src_v[f]])
                msgs[b][pl.ds(g * 16, 16)] = zg * ew_v[f]
            pltpu.async_copy(msgs[b], agg_sh.at[dst_v.at[j]], sems[b],
                             add=True)
        return carry

    lax.fori_loop(0, CPW // 2, body, 0)
    for b in range(2):
        pltpu.make_async_copy(msgs[b], agg_sh.at[dst_v.at[0]], sems[b]).wait()
    plsc.subcore_barrier()
    pltpu.sync_copy(agg_sh.at[pl.ds(s * NPT, NPT)],
                    out.at[pl.ds(c * NP + s * NPT, NPT)])


@functools.partial(
    pl.kernel,
    out_type=jax.ShapeDtypeStruct((NP,), F32),
    mesh=_MESH,
    compiler_params=_SC_PARAMS,
    scratch_types=[
        pltpu.VMEM((NPW,), F32),
        pltpu.VMEM((NPW,), F32),
        pltpu.VMEM((NPW,), F32),
        pltpu.VMEM((NPW,), F32),
        pltpu.VMEM((NPW,), F32),
        pltpu.VMEM((16,), F32),
        pltpu.VMEM((16,), F32),
        pltpu.VMEM((16,), F32),
    ],
)
def _sc_fin(p2, zf, disf, b2s, wls, bls, res_o, a0, a1, zv, dv, rv,
            b2v, wlv, blv):
    c = lax.axis_index("c")
    s = lax.axis_index("s")
    w = s * NC + c
    nb = w * NPW
    pltpu.sync_copy(p2.at[pl.ds(nb, NPW)], a0)
    pltpu.sync_copy(p2.at[pl.ds(NP + nb, NPW)], a1)
    pltpu.sync_copy(zf.at[pl.ds(nb, NPW)], zv)
    pltpu.sync_copy(disf.at[pl.ds(nb, NPW)], dv)
    pltpu.sync_copy(b2s, b2v)
    pltpu.sync_copy(wls, wlv)
    pltpu.sync_copy(bls, blv)
    b2x = b2v[pl.ds(0, 16)]
    wlx = wlv[pl.ds(0, 16)]
    blx = blv[pl.ds(0, 16)]
    for i in range(NPW // 16):
        f = pl.ds(i * 16, 16)
        o2 = dv[f] * (a0[f] + a1[f] + zv[f]) + b2x
        t = o2 * wlx + blx
        rv[f] = 1.0 / (1.0 + jnp.exp(-t))
    pltpu.sync_copy(rv, res_o.at[pl.ds(nb, NPW)])


def kernel(x, edge_index, edge_attr, W1, b1, W2, b2, Wl, bl):
    src = edge_index[0]
    dst = edge_index[1]
    npad = EP - E
    pad_idx = (jnp.arange(npad, dtype=jnp.int32) * 37) % N
    srcp = jnp.concatenate([src, pad_idx])
    dstp = jnp.concatenate([dst, pad_idx])
    ewp = jnp.concatenate([edge_attr, jnp.zeros((npad,), F32)])
    dst2d = dstp.reshape(EP // CH, CH)
    ew2d = ewp.reshape(EP // CH, CH)

    xw = _matmul(x, W1)
    xwf = jnp.pad(xw, ((0, NP - N), (0, 0))).reshape(NP * 4)

    zeros1 = jnp.zeros((NP,), F32)
    zeros4 = jnp.zeros((NP * 4,), F32)
    b1rep = jnp.tile(b1, 4)
    w2rep = jnp.tile(W2[:, 0], 4)
    b2s = jnp.full((16,), b2[0], F32)
    wls = jnp.full((16,), Wl[0, 0], F32)
    bls = jnp.full((16,), bl[0], F32)

    degp = _sc_deg(dst2d, ew2d, zeros1)
    dis, yf = _sc_dis_y(degp, xwf)
    p1f = _sc_edge1(srcp, ewp, dstp, yf, zeros4)
    z = _sc_hz(p1f, yf, dis, b1rep, w2rep)
    p2 = _sc_edge2(srcp, ewp, dst2d, z, zeros1)
    res = _sc_fin(p2, z, dis, b2s, wls, bls)
    return res[:N, None]
